# Initial kernel scaffold; baseline (speedup 1.0000x reference)
import jax
import jax.numpy as jnp
from jax.experimental import pallas as pl


def kernel(x_batch, adj, k1, k2, Wd1, bd1, Wd2, bd2):
    raise NotImplementedError("write your pallas kernel here")


# speedup vs baseline: 1.1207x; 1.1207x over previous
import jax
import jax.numpy as jnp
from jax.experimental import pallas as pl
from jax.experimental.pallas import tpu as pltpu

B, N, F = 8, 128, 128
C1, C2 = 256, 128


def _body(x_ref, adj_ref, k1_ref, k2_ref, Wd1_ref, bd1_ref, Wd2_ref, bd2_ref,
          out_ref):
    x = x_ref[...]
    adj = adj_ref[...]

    xs = jnp.sum(x, axis=-1)
    t = adj * xs[:, None, :]
    v1 = jnp.sum(t, axis=-1)
    v2 = jnp.sum(t * adj, axis=-1)
    s0 = jnp.sum(xs, axis=-1)

    k1 = k1_ref[...]
    arg1 = (k1[:, 0][:, None, None] * v2[None]
            + k1[:, 1][:, None, None] * v1[None]
            + k1[:, 2][:, None, None] * s0[None, :, None])
    h1s = jnp.sum(jnp.tanh(arg1), axis=0)

    w1 = jnp.sum(adj * h1s[:, None, :], axis=-1)
    t0 = jnp.sum(h1s, axis=-1)

    k2 = k2_ref[...]
    arg2 = (k2[:, 0][:, None, None] * w1[None]
            + k2[:, 1][:, None, None] * t0[None, :, None])
    pooled = jnp.sum(jnp.tanh(arg2), axis=-1)
    pooled = pooled.T

    d1 = jnp.tanh(
        jax.lax.dot_general(pooled, Wd1_ref[...],
                            (((1,), (0,)), ((), ())),
                            precision=jax.lax.Precision.HIGHEST,
                            preferred_element_type=jnp.float32)
        + bd1_ref[...][None, :])
    logits = jax.lax.dot_general(d1, Wd2_ref[...],
                                 (((1,), (0,)), ((), ())),
                                 precision=jax.lax.Precision.HIGHEST,
                                 preferred_element_type=jnp.float32) \
        + bd2_ref[...][None, :]

    m = jnp.max(logits, axis=-1, keepdims=True)
    e = jnp.exp(logits - m)
    out_ref[...] = e / jnp.sum(e, axis=-1, keepdims=True)


def kernel(x_batch, adj, k1, k2, Wd1, bd1, Wd2, bd2):
    return pl.pallas_call(
        _body,
        out_shape=jax.ShapeDtypeStruct((B, 10), jnp.float32),
    )(x_batch, adj, k1, k2, Wd1, bd1, Wd2, bd2)


# speedup vs baseline: 5.8787x; 5.2458x over previous
import jax
import jax.numpy as jnp
from jax.experimental import pallas as pl

B, N, F = 8, 128, 128
C1, C2 = 256, 128
HI = jax.lax.Precision.HIGHEST


def _dot(a, b):
    return jax.lax.dot_general(a, b, (((1,), (0,)), ((), ())),
                               precision=HI,
                               preferred_element_type=jnp.float32)


def _tanh(x):
    x = jnp.clip(x, -7.90531110763549805, 7.90531110763549805)
    x2 = x * x
    p = 2.00018790482477e-13 + x2 * -2.76076847742355e-16
    p = -8.60467152213735e-11 + x2 * p
    p = 5.12229709037114e-08 + x2 * p
    p = 1.48572235717979e-05 + x2 * p
    p = 6.37261928875436e-04 + x2 * p
    p = 4.89352455891786e-03 + x2 * p
    p = x * p
    q = 1.19825839466702e-06
    q = 1.18534705686654e-04 + x2 * q
    q = 2.26843463243900e-03 + x2 * q
    q = 4.89352518554385e-03 + x2 * q
    return p / q


def _body(x_ref, adj_ref, k1_ref, k2_ref, Wd1_ref, bd1_ref, Wd2_ref, bd2_ref,
          out_ref):
    x = x_ref[...]
    adj2d = adj_ref[...].reshape(B * N, N)

    xs = jnp.sum(x, axis=-1)
    Xs = xs.T

    Gb = _dot(adj2d, Xs).reshape(B, N, B)
    Gq = _dot(adj2d * adj2d, Xs).reshape(B, N, B)
    bidx = jax.lax.broadcasted_iota(jnp.int32, (B, N, B), 0)
    lidx = jax.lax.broadcasted_iota(jnp.int32, (B, N, B), 2)
    sel = bidx == lidx
    zero = jnp.zeros((), jnp.float32)
    v1 = jnp.sum(jnp.where(sel, Gb, zero), axis=-1)
    v2 = jnp.sum(jnp.where(sel, Gq, zero), axis=-1)
    s0 = jnp.sum(xs, axis=-1, keepdims=True)
    s0b = jnp.broadcast_to(s0, (B, N))

    M = jnp.concatenate([v2.reshape(1, B * N),
                         v1.reshape(1, B * N),
                         s0b.reshape(1, B * N)], axis=0)
    T1 = _tanh(_dot(k1_ref[...], M))
    h1s = _dot(jnp.ones((1, C1), jnp.float32), T1).reshape(B, N)

    Hs = h1s.T
    G2 = _dot(adj2d, Hs).reshape(B, N, B)
    w1 = jnp.sum(jnp.where(sel, G2, zero), axis=-1)
    t0 = jnp.sum(h1s, axis=-1, keepdims=True)
    t0b = jnp.broadcast_to(t0, (B, N))

    M3 = jnp.concatenate([w1.reshape(1, B * N),
                          t0b.reshape(1, B * N)], axis=0)
    T2 = _tanh(_dot(k2_ref[...], M3))

    rowb = jax.lax.broadcasted_iota(jnp.int32, (B * N, B), 0) // N
    colb = jax.lax.broadcasted_iota(jnp.int32, (B * N, B), 1)
    S = (rowb == colb).astype(jnp.float32)
    pooled = _dot(T2, S).T

    d1 = _tanh(_dot(pooled, Wd1_ref[...]) + bd1_ref[...][None, :])
    logits = _dot(d1, Wd2_ref[...]) + bd2_ref[...][None, :]

    m = jnp.max(logits, axis=-1, keepdims=True)
    e = jnp.exp(logits - m)
    out_ref[...] = e / jnp.sum(e, axis=-1, keepdims=True)


def kernel(x_batch, adj, k1, k2, Wd1, bd1, Wd2, bd2):
    return pl.pallas_call(
        _body,
        out_shape=jax.ShapeDtypeStruct((B, 10), jnp.float32),
    )(x_batch, adj, k1, k2, Wd1, bd1, Wd2, bd2)
